# baseline (speedup 1.0000x reference)
import functools

import jax
import jax.numpy as jnp
from jax import lax
from jax.experimental import pallas as pl
from jax.experimental.pallas import tpu as pltpu
from jax.experimental.pallas import tpu_sc as plsc

VOCAB = 100000
IMG = 2048
HID = 128
TOTAL = VOCAB + IMG

NTILES = 32
PER_TILE = 3200
CHUNK = 128
NCHUNK = PER_TILE // CHUNK

BC = 4096
NR = 3

RS_PAD = 25 * BC
WH_COVER = 102016

_EMB_CHUNKS = [(c * BC, BC) for c in range(VOCAB // BC)] + [
    ((VOCAB // BC) * BC, VOCAB - (VOCAB // BC) * BC)]
_WH_CHUNKS = [(c * BC, BC) for c in range(WH_COVER // BC)] + [
    ((WH_COVER // BC) * BC, WH_COVER - (WH_COVER // BC) * BC)]


def _pass1_body(emb_hbm, wh_hbm, wo_ref, rs_ref, v_ref, eb, wb, esem, wsem):
    def efire(c):
        off, n = _EMB_CHUNKS[c]
        d = pltpu.make_async_copy(
            emb_hbm.at[pl.ds(off, n), :], eb.at[c % NR, pl.ds(0, n), :],
            esem.at[c % NR])
        d.start()
        return d

    def wfire(c):
        off, n = _WH_CHUNKS[c]
        d = pltpu.make_async_copy(
            wh_hbm.at[:, pl.ds(off, n)], wb.at[c % NR, :, pl.ds(0, n)],
            wsem.at[c % NR])
        d.start()
        return d

    nch = len(_EMB_CHUNKS)
    eh = [None] * nch
    wh = [None] * nch
    for c in range(NR):
        eh[c] = efire(c)
        wh[c] = wfire(c)
    wo = wo_ref[...]
    for c in range(nch):
        eoff, _ = _EMB_CHUNKS[c]
        eh[c].wait()
        rs_ref[0, pl.ds(eoff, BC)] = jnp.sum(eb[c % NR], axis=1)
        if c + NR < nch:
            eh[c + NR] = efire(c + NR)
        woff, wn = _WH_CHUNKS[c]
        wh[c].wait()
        v_ref[0, pl.ds(woff, wn)] = jnp.dot(
            wo, wb[c % NR][:, :wn],
            preferred_element_type=jnp.float32)[0]
        if c + NR < nch:
            wh[c + NR] = wfire(c + NR)


def _sc_gather_dot(idx_hbm, vw_hbm, table_hbm, out_hbm,
                   idx_v, vw_v, rows_v, acc_v, sem):
    wid = lax.axis_index("s") * 2 + lax.axis_index("c")
    pltpu.sync_copy(idx_hbm.at[wid], idx_v)
    pltpu.sync_copy(vw_hbm.at[wid], vw_v)
    copies = [
        pltpu.async_copy(table_hbm.at[idx_v.at[j]], rows_v.at[j], sem)
        for j in range(NCHUNK)
    ]
    for c in copies:
        c.wait()

    acc = jnp.zeros((16,), jnp.float32)
    for j in range(NCHUNK):
        def body(g, a, j=j):
            vals = rows_v[j, pl.ds(g * 16, 16)]
            w = vw_v[j, pl.ds(g * 16, 16)]
            return a + vals * w
        acc = lax.fori_loop(0, CHUNK // 16, body, acc)
    acc_v[...] = acc
    pltpu.sync_copy(acc_v, out_hbm.at[wid])


def _final_body(p_ref, vi_ref, img_ref, ws_ref, wo_ref, bh_ref, bo_ref,
                o_ref):
    word = jnp.sum(p_ref[...])
    imgv = img_ref[...]
    img = jnp.sum(vi_ref[...] * imgv[:, :WH_COVER - VOCAB])
    sliver_v = jnp.dot(wo_ref[...], ws_ref[...],
                       preferred_element_type=jnp.float32)
    img = img + jnp.sum(sliver_v * imgv[:, WH_COVER - VOCAB:])
    c = jnp.sum(wo_ref[...] * bh_ref[...]) + bo_ref[0, 0]
    x = word + img + c
    o_ref[...] = (1.0 / (1.0 + jnp.exp(-x))).reshape(1, 1)


def kernel(word_inputs, image_inputs, emb_table, W_h, b_h, W_o, b_o):
    rowsum2d, v2d = pl.pallas_call(
        _pass1_body,
        in_specs=[
            pl.BlockSpec(memory_space=pl.ANY),
            pl.BlockSpec(memory_space=pl.ANY),
            pl.BlockSpec((1, HID), lambda: (0, 0)),
        ],
        out_specs=[
            pl.BlockSpec((1, RS_PAD), lambda: (0, 0)),
            pl.BlockSpec((1, WH_COVER), lambda: (0, 0)),
        ],
        out_shape=[
            jax.ShapeDtypeStruct((1, RS_PAD), jnp.float32),
            jax.ShapeDtypeStruct((1, WH_COVER), jnp.float32),
        ],
        scratch_shapes=[
            pltpu.VMEM((NR, BC, HID), jnp.float32),
            pltpu.VMEM((NR, HID, BC), jnp.float32),
            pltpu.SemaphoreType.DMA((NR,)),
            pltpu.SemaphoreType.DMA((NR,)),
        ],
    )(emb_table, W_h, W_o)

    rowsum = rowsum2d.reshape(RS_PAD)
    v = v2d.reshape(WH_COVER)

    pad = NTILES * PER_TILE - VOCAB
    idx_pad = jnp.concatenate(
        [word_inputs.astype(jnp.int32), jnp.zeros((pad,), jnp.int32)])
    vw_pad = jnp.concatenate([v[:VOCAB], jnp.zeros((pad,), jnp.float32)])
    v_img = v[VOCAB:].reshape(1, WH_COVER - VOCAB)
    w_sliver = W_h[:, WH_COVER:]
    idx3d = idx_pad.reshape(NTILES, NCHUNK, CHUNK)
    vw3d = vw_pad.reshape(NTILES, NCHUNK, CHUNK)

    mesh = plsc.VectorSubcoreMesh(core_axis_name="c", subcore_axis_name="s")
    partials = functools.partial(
        pl.kernel,
        mesh=mesh,
        out_type=jax.ShapeDtypeStruct((NTILES, 16), jnp.float32),
        scratch_types=[
            pltpu.VMEM((NCHUNK, CHUNK), jnp.int32),
            pltpu.VMEM((NCHUNK, CHUNK), jnp.float32),
            pltpu.VMEM((NCHUNK, CHUNK), jnp.float32),
            pltpu.VMEM((16,), jnp.float32),
            pltpu.SemaphoreType.DMA,
        ],
    )(_sc_gather_dot)(idx3d, vw3d, rowsum)

    out2d = pl.pallas_call(
        _final_body,
        out_shape=jax.ShapeDtypeStruct((1, 1), jnp.float32),
    )(partials, v_img, image_inputs.reshape(1, IMG), w_sliver, W_o,
      b_h.reshape(1, HID), b_o.reshape(1, 1))
    return out2d.reshape(1)
